# baseline (speedup 1.0000x reference)
import jax
import jax.numpy as jnp
from jax import lax
from jax.experimental import pallas as pl
from jax.experimental.pallas import tpu as pltpu
from jax.experimental.pallas import tpu_sc as plsc

_VOCAB = 32000
_HIDDEN = 768
_SEQ = 64
_BATCH = 4096
_NC, _NS, _L = 2, 16, 16
_NW = _NC * _NS
_TOK = _BATCH * _SEQ
_TPW = _TOK // _NW
_C = 32
_NCHUNK = _TPW // _C
_NVREG = _HIDDEN // _L
_SCALE = float(_HIDDEN) ** 0.5


def _body(ids_hbm, table_hbm, pos_hbm, out_hbm, idx_v, rows_v, pos_v, gsem):
    wid = lax.axis_index("s") * _NC + lax.axis_index("c")
    base = wid * _TPW
    pltpu.sync_copy(pos_hbm, pos_v)
    pltpu.sync_copy(ids_hbm.at[pl.ds(base, _TPW)], idx_v)

    @pl.loop(0, _NCHUNK)
    def _chunk(k):
        pltpu.async_copy(
            table_hbm.at[idx_v.at[pl.ds(k * _C, _C)]], rows_v, gsem
        ).wait()
        p = (k % 2) * _C

        @pl.loop(0, _C)
        def _row(r):
            for j in range(_NVREG):
                x = rows_v[r, pl.ds(j * _L, _L)]
                y = pos_v[p + r, pl.ds(j * _L, _L)]
                rows_v[r, pl.ds(j * _L, _L)] = x * _SCALE + y

        pltpu.sync_copy(rows_v, out_hbm.at[pl.ds(base + k * _C, _C)])


def kernel(input_ids, token_embedding, position_embedding):
    ids_flat = input_ids.reshape(-1).astype(jnp.int32)
    run = pl.kernel(
        _body,
        out_type=jax.ShapeDtypeStruct((_TOK, _HIDDEN), jnp.float32),
        mesh=plsc.VectorSubcoreMesh(core_axis_name="c", subcore_axis_name="s"),
        scratch_types=[
            pltpu.VMEM((_TPW,), jnp.int32),
            pltpu.VMEM((_C, _HIDDEN), jnp.float32),
            pltpu.VMEM((_SEQ, _HIDDEN), jnp.float32),
            pltpu.SemaphoreType.DMA,
        ],
    )
    out = run(ids_flat, token_embedding, position_embedding)
    return out.reshape(_BATCH, _SEQ, _HIDDEN)


# speedup vs baseline: 1.9351x; 1.9351x over previous
import jax
import jax.numpy as jnp
from jax import lax
from jax.experimental import pallas as pl
from jax.experimental.pallas import tpu as pltpu
from jax.experimental.pallas import tpu_sc as plsc

_VOCAB = 32000
_HIDDEN = 768
_SEQ = 64
_BATCH = 4096
_NC, _NS, _L = 2, 16, 16
_NW = _NC * _NS
_TOK = _BATCH * _SEQ
_TPW = _TOK // _NW
_C = 16
_NBUF = 4
_D = 2
_NCHUNK = _TPW // _C
_NVREG = _HIDDEN // _L
_SCALE = float(_HIDDEN) ** 0.5


def _body(ids_hbm, table_hbm, pos_hbm, out_hbm, idx_v, rows0, rows1, rows2,
          rows3, pos_v, g0, g1, g2, g3, s0, s1, s2, s3):
    rows = (rows0, rows1, rows2, rows3)
    gsem = (g0, g1, g2, g3)
    ssem = (s0, s1, s2, s3)
    wid = lax.axis_index("s") * _NC + lax.axis_index("c")
    base = wid * _TPW
    pltpu.sync_copy(pos_hbm, pos_v)
    pltpu.sync_copy(ids_hbm.at[pl.ds(base, _TPW)], idx_v)

    def start_gather(k, b):
        pltpu.async_copy(
            table_hbm.at[idx_v.at[pl.ds(k * _C, _C)]], rows[b], gsem[b])

    def drain_gather(k, b):
        pltpu.make_async_copy(
            table_hbm.at[idx_v.at[pl.ds(k * _C, _C)]], rows[b], gsem[b]).wait()

    def out_slice(k):
        return out_hbm.at[pl.ds(base + k * _C, _C)]

    for k in range(_D):
        start_gather(k, k % _NBUF)

    @pl.loop(0, _NCHUNK, step=_NBUF)
    def _ring(k0):
        for b in range(_NBUF):
            k = k0 + b
            nb = (b + _D) % _NBUF

            @pl.when(k + _D - _NBUF >= 0)
            def _():
                pltpu.make_async_copy(
                    rows[nb], out_slice(k + _D - _NBUF), ssem[nb]).wait()

            @pl.when(k + _D < _NCHUNK)
            def _():
                start_gather(k + _D, nb)

            drain_gather(k, b)
            p = (k % (_SEQ // _C)) * _C

            @pl.loop(0, _C)
            def _row(r):
                for j in range(_NVREG):
                    x = rows[b][r, pl.ds(j * _L, _L)]
                    y = pos_v[p + r, pl.ds(j * _L, _L)]
                    rows[b][r, pl.ds(j * _L, _L)] = x * _SCALE + y

            pltpu.async_copy(rows[b], out_slice(k), ssem[b])

    for k in range(_NCHUNK - _NBUF + _D, _NCHUNK):
        b = k % _NBUF
        pltpu.make_async_copy(rows[b], out_slice(k), ssem[b]).wait()


def kernel(input_ids, token_embedding, position_embedding):
    ids_flat = input_ids.reshape(-1).astype(jnp.int32)
    run = pl.kernel(
        _body,
        out_type=jax.ShapeDtypeStruct((_TOK, _HIDDEN), jnp.float32),
        mesh=plsc.VectorSubcoreMesh(core_axis_name="c", subcore_axis_name="s"),
        scratch_types=[
            pltpu.VMEM((_TPW,), jnp.int32),
            pltpu.VMEM((_C, _HIDDEN), jnp.float32),
            pltpu.VMEM((_C, _HIDDEN), jnp.float32),
            pltpu.VMEM((_C, _HIDDEN), jnp.float32),
            pltpu.VMEM((_C, _HIDDEN), jnp.float32),
            pltpu.VMEM((_SEQ, _HIDDEN), jnp.float32),
            pltpu.SemaphoreType.DMA,
            pltpu.SemaphoreType.DMA,
            pltpu.SemaphoreType.DMA,
            pltpu.SemaphoreType.DMA,
            pltpu.SemaphoreType.DMA,
            pltpu.SemaphoreType.DMA,
            pltpu.SemaphoreType.DMA,
            pltpu.SemaphoreType.DMA,
        ],
    )
    out = run(ids_flat, token_embedding, position_embedding)
    return out.reshape(_BATCH, _SEQ, _HIDDEN)
